# Initial kernel scaffold; baseline (speedup 1.0000x reference)
import jax
import jax.numpy as jnp
from jax.experimental import pallas as pl


def kernel(x, edge_index, W_lin, b_lin, W_gc, b_gc):
    raise NotImplementedError("write your pallas kernel here")


# speedup vs baseline: 12.1990x; 12.1990x over previous
import functools

import jax
import jax.numpy as jnp
from jax import lax
from jax.experimental import pallas as pl
from jax.experimental.pallas import tpu as pltpu
from jax.experimental.pallas import tpu_sc as plsc

ROW_BLK = 2000
EDGE_CHUNK = 1000
NS = 16
H_HALF = 16


def _proj_body(x_ref, wl_ref, bl_ref, wg_ref, za_ref, zb_ref):
    z1 = jnp.dot(x_ref[...], wl_ref[...], preferred_element_type=jnp.float32)
    z1 = z1 + bl_ref[...]
    z2 = jnp.dot(z1, wg_ref[...], preferred_element_type=jnp.float32)
    za_ref[...] = z2[:, :H_HALF]
    zb_ref[...] = z2[:, H_HALF:]


def _project(x, W_lin, b_lin, W_gc):
    n, d = x.shape
    h1 = W_lin.shape[1]
    h2 = W_gc.shape[1]
    grid = n // ROW_BLK
    return pl.pallas_call(
        _proj_body,
        grid=(grid,),
        in_specs=[
            pl.BlockSpec((ROW_BLK, d), lambda i: (i, 0)),
            pl.BlockSpec((d, h1), lambda i: (0, 0)),
            pl.BlockSpec((1, h1), lambda i: (0, 0)),
            pl.BlockSpec((h1, h2), lambda i: (0, 0)),
        ],
        out_specs=[
            pl.BlockSpec((ROW_BLK, H_HALF), lambda i: (i, 0)),
            pl.BlockSpec((ROW_BLK, H_HALF), lambda i: (i, 0)),
        ],
        out_shape=[
            jax.ShapeDtypeStruct((n, H_HALF), jnp.float32),
            jax.ShapeDtypeStruct((n, H_HALF), jnp.float32),
        ],
    )(x, W_lin, b_lin.reshape(1, h1), W_gc)


def _sc_aggregate(z2a, z2b, src, dst, zeros_agg, zeros_deg):
    n = z2a.shape[0]
    e = src.shape[0]
    ept = e // NS
    nchunks = ept // EDGE_CHUNK
    half = nchunks // 2
    mesh = plsc.VectorSubcoreMesh(core_axis_name="c", subcore_axis_name="s")

    @functools.partial(
        pl.kernel,
        out_type=[
            jax.ShapeDtypeStruct((n, H_HALF), jnp.float32),
            jax.ShapeDtypeStruct((n, H_HALF), jnp.float32),
            jax.ShapeDtypeStruct((2, n), jnp.float32),
        ],
        mesh=mesh,
        compiler_params=pltpu.CompilerParams(use_tc_tiling_on_sc=False),
        scratch_types=[
            pltpu.VMEM_SHARED((n, H_HALF), jnp.float32),
            pltpu.VMEM_SHARED((n,), jnp.float32),
            pltpu.VMEM((EDGE_CHUNK,), jnp.int32),
            pltpu.VMEM((EDGE_CHUNK,), jnp.int32),
            pltpu.VMEM((EDGE_CHUNK, H_HALF), jnp.float32),
            pltpu.VMEM((EDGE_CHUNK,), jnp.float32),
            pltpu.SemaphoreType.DMA,
        ],
    )
    def body(za_hbm, zb_hbm, src_hbm, dst_hbm, zag_hbm, zdg_hbm,
             agg_a_out, agg_b_out, deg_out,
             agg_sh, deg_sh, srcb, dstb, rowsb, onesb, sem):
        c = lax.axis_index("c")
        s = lax.axis_index("s")

        def fill_ones(i, carry):
            onesb[pl.ds(i * 16, 16)] = jnp.full((16,), 1.0, jnp.float32)
            return carry
        lax.fori_loop(0, EDGE_CHUNK // 16, fill_ones, 0)
        if EDGE_CHUNK % 16:
            onesb[pl.ds(EDGE_CHUNK - 16, 16)] = jnp.full((16,), 1.0, jnp.float32)

        @pl.when(s == 0)
        def _init():
            pltpu.sync_copy(zag_hbm, agg_sh)
            pltpu.sync_copy(zdg_hbm, deg_sh)

        plsc.subcore_barrier()

        tile_base = s * ept

        def make_chunk(table_hbm):
            def chunk(k, carry):
                eb = tile_base + k * EDGE_CHUNK
                pltpu.sync_copy(src_hbm.at[pl.ds(eb, EDGE_CHUNK)], srcb)
                pltpu.sync_copy(dst_hbm.at[pl.ds(eb, EDGE_CHUNK)], dstb)
                pltpu.async_copy(table_hbm.at[srcb], rowsb, sem).wait()
                pltpu.sync_copy(rowsb, agg_sh.at[dstb], add=True)
                do_deg = jnp.where(c == 0, k < half, k >= half)

                @pl.when(do_deg)
                def _():
                    pltpu.sync_copy(onesb, deg_sh.at[dstb], add=True)
                return carry
            return chunk

        @pl.when(c == 0)
        def _loop_a():
            lax.fori_loop(0, nchunks, make_chunk(za_hbm), 0)

        @pl.when(c == 1)
        def _loop_b():
            lax.fori_loop(0, nchunks, make_chunk(zb_hbm), 0)

        plsc.subcore_barrier()

        @pl.when(s == 0)
        def _writeback():
            @pl.when(c == 0)
            def _():
                pltpu.sync_copy(agg_sh, agg_a_out)
                pltpu.sync_copy(deg_sh, deg_out.at[0])

            @pl.when(c == 1)
            def _():
                pltpu.sync_copy(agg_sh, agg_b_out)
                pltpu.sync_copy(deg_sh, deg_out.at[1])

    return body(z2a, z2b, src, dst, zeros_agg, zeros_deg)


def _finish_body(a_ref, b_ref, d0_ref, d1_ref, bg_ref, o_ref):
    agg = jnp.concatenate([a_ref[...], b_ref[...]], axis=1)
    deg = jnp.maximum(d0_ref[...] + d1_ref[...], 1.0)
    o_ref[...] = jnp.maximum(agg / deg + bg_ref[...], 0.0)


def _finish(agg_a, agg_b, d0, d1, b_gc):
    n = agg_a.shape[0]
    h2 = 2 * H_HALF
    grid = n // ROW_BLK
    return pl.pallas_call(
        _finish_body,
        grid=(grid,),
        in_specs=[
            pl.BlockSpec((ROW_BLK, H_HALF), lambda i: (i, 0)),
            pl.BlockSpec((ROW_BLK, H_HALF), lambda i: (i, 0)),
            pl.BlockSpec((ROW_BLK, 1), lambda i: (i, 0)),
            pl.BlockSpec((ROW_BLK, 1), lambda i: (i, 0)),
            pl.BlockSpec((1, h2), lambda i: (0, 0)),
        ],
        out_specs=pl.BlockSpec((ROW_BLK, h2), lambda i: (i, 0)),
        out_shape=jax.ShapeDtypeStruct((n, h2), jnp.float32),
    )(agg_a, agg_b, d0, d1, b_gc.reshape(1, h2))


def kernel(x, edge_index, W_lin, b_lin, W_gc, b_gc):
    n = x.shape[0]
    src = edge_index[0].astype(jnp.int32)
    dst = edge_index[1].astype(jnp.int32)
    z2a, z2b = _project(x, W_lin, b_lin, W_gc)
    zeros_agg = jnp.zeros((n, H_HALF), jnp.float32)
    zeros_deg = jnp.zeros((n,), jnp.float32)
    agg_a, agg_b, deg_pair = _sc_aggregate(z2a, z2b, src, dst, zeros_agg, zeros_deg)
    d0 = deg_pair[0].reshape(n, 1)
    d1 = deg_pair[1].reshape(n, 1)
    return _finish(agg_a, agg_b, d0, d1, b_gc)


# speedup vs baseline: 13.8918x; 1.1388x over previous
import functools

import jax
import jax.numpy as jnp
from jax import lax
from jax.experimental import pallas as pl
from jax.experimental.pallas import tpu as pltpu
from jax.experimental.pallas import tpu_sc as plsc

ROW_BLK = 2000
EDGE_CHUNK = 800
NODE_CHUNK = 800
NS = 16
H_HALF = 16


def _proj_body(x_ref, wl_ref, bl_ref, wg_ref, za_ref, zb_ref):
    z1 = jnp.dot(x_ref[...], wl_ref[...], preferred_element_type=jnp.float32)
    z1 = z1 + bl_ref[...]
    z2 = jnp.dot(z1, wg_ref[...], preferred_element_type=jnp.float32)
    za_ref[...] = z2[:, :H_HALF]
    zb_ref[...] = z2[:, H_HALF:]


def _project(x, W_lin, b_lin, W_gc):
    n, d = x.shape
    h1 = W_lin.shape[1]
    h2 = W_gc.shape[1]
    grid = n // ROW_BLK
    return pl.pallas_call(
        _proj_body,
        grid=(grid,),
        in_specs=[
            pl.BlockSpec((ROW_BLK, d), lambda i: (i, 0)),
            pl.BlockSpec((d, h1), lambda i: (0, 0)),
            pl.BlockSpec((1, h1), lambda i: (0, 0)),
            pl.BlockSpec((h1, h2), lambda i: (0, 0)),
        ],
        out_specs=[
            pl.BlockSpec((ROW_BLK, H_HALF), lambda i: (i, 0)),
            pl.BlockSpec((ROW_BLK, H_HALF), lambda i: (i, 0)),
        ],
        out_shape=[
            jax.ShapeDtypeStruct((n, H_HALF), jnp.float32),
            jax.ShapeDtypeStruct((n, H_HALF), jnp.float32),
        ],
    )(x, W_lin, b_lin.reshape(1, h1), W_gc)


def _sc_aggregate(z2a, z2b, src, dst, b_gc):
    n = z2a.shape[0]
    e = src.shape[0]
    h2 = b_gc.shape[0]
    ept = e // NS
    nchunks = ept // EDGE_CHUNK
    node_chunks = n // NODE_CHUNK
    ncpt = node_chunks // NS
    ncrem = node_chunks - ncpt * NS
    mesh = plsc.VectorSubcoreMesh(core_axis_name="c", subcore_axis_name="s")

    @functools.partial(
        pl.kernel,
        out_type=jax.ShapeDtypeStruct((n, h2), jnp.float32),
        mesh=mesh,
        compiler_params=pltpu.CompilerParams(use_tc_tiling_on_sc=False),
        scratch_types=[
            pltpu.VMEM_SHARED((n, H_HALF), jnp.float32),
            pltpu.VMEM_SHARED((n,), jnp.float32),
            pltpu.VMEM((EDGE_CHUNK,), jnp.int32),
            pltpu.VMEM((EDGE_CHUNK,), jnp.int32),
            pltpu.VMEM((NODE_CHUNK, H_HALF), jnp.float32),
            pltpu.VMEM((EDGE_CHUNK,), jnp.float32),
            pltpu.VMEM((NODE_CHUNK,), jnp.float32),
            pltpu.VMEM((NODE_CHUNK,), jnp.float32),
            pltpu.VMEM((32,), jnp.float32),
            pltpu.SemaphoreType.DMA,
        ],
    )
    def body(za_hbm, zb_hbm, src_hbm, dst_hbm, bgc_hbm, out_hbm,
             agg_sh, deg_sh, srcb, dstb, rowsb, onesb, degb, recb, bgcb, sem):
        c = lax.axis_index("c")
        s = lax.axis_index("s")

        def fill16(i, carry):
            onesb[pl.ds(i * 16, 16)] = jnp.full((16,), 1.0, jnp.float32)
            degb[pl.ds(i * 16, 16)] = jnp.zeros((16,), jnp.float32)
            return carry
        lax.fori_loop(0, EDGE_CHUNK // 16, fill16, 0)

        def zero_rows(i, carry):
            rowsb[i] = jnp.zeros((H_HALF,), jnp.float32)
            return carry
        lax.fori_loop(0, NODE_CHUNK, zero_rows, 0)

        pltpu.sync_copy(bgc_hbm, bgcb)

        for j in range(ncpt):
            k = s + NS * j
            pltpu.sync_copy(rowsb, agg_sh.at[pl.ds(k * NODE_CHUNK, NODE_CHUNK)])
            pltpu.sync_copy(degb, deg_sh.at[pl.ds(k * NODE_CHUNK, NODE_CHUNK)])

        @pl.when(s < ncrem)
        def _zero_rem():
            k = ncpt * NS + s
            pltpu.sync_copy(rowsb, agg_sh.at[pl.ds(k * NODE_CHUNK, NODE_CHUNK)])
            pltpu.sync_copy(degb, deg_sh.at[pl.ds(k * NODE_CHUNK, NODE_CHUNK)])

        plsc.subcore_barrier()

        tile_base = s * ept

        def make_chunk(table_hbm):
            def chunk(k, carry):
                eb = tile_base + k * EDGE_CHUNK
                pltpu.sync_copy(src_hbm.at[pl.ds(eb, EDGE_CHUNK)], srcb)
                pltpu.sync_copy(dst_hbm.at[pl.ds(eb, EDGE_CHUNK)], dstb)
                pltpu.async_copy(table_hbm.at[srcb], rowsb, sem).wait()
                pltpu.sync_copy(rowsb, agg_sh.at[dstb], add=True)
                pltpu.sync_copy(onesb, deg_sh.at[dstb], add=True)
                return carry
            return chunk

        @pl.when(c == 0)
        def _loop_a():
            lax.fori_loop(0, nchunks, make_chunk(za_hbm), 0)

        @pl.when(c == 1)
        def _loop_b():
            lax.fori_loop(0, nchunks, make_chunk(zb_hbm), 0)

        plsc.subcore_barrier()

        bias_a = bgcb[pl.ds(0, H_HALF)]
        bias_b = bgcb[pl.ds(H_HALF, H_HALF)]

        def norm_chunk(k):
            r0 = k * NODE_CHUNK
            pltpu.sync_copy(agg_sh.at[pl.ds(r0, NODE_CHUNK)], rowsb)
            pltpu.sync_copy(deg_sh.at[pl.ds(r0, NODE_CHUNK)], degb)

            def recips(i, carry):
                d16 = degb[pl.ds(i * 16, 16)]
                recb[pl.ds(i * 16, 16)] = 1.0 / jnp.maximum(d16, 1.0)
                return carry
            lax.fori_loop(0, NODE_CHUNK // 16, recips, 0)

            def norm_rows(bias):
                def fn(i, carry):
                    rec16 = recb[pl.ds(i * 16, 16)]
                    base = i * 16
                    for j in range(16):
                        rowsb[base + j] = jnp.maximum(
                            rowsb[base + j] * rec16[j] + bias, 0.0)
                    return carry
                return fn

            @pl.when(c == 0)
            def _():
                lax.fori_loop(0, NODE_CHUNK // 16, norm_rows(bias_a), 0)
                pltpu.sync_copy(rowsb, out_hbm.at[pl.ds(r0, NODE_CHUNK), pl.ds(0, H_HALF)])

            @pl.when(c == 1)
            def _():
                lax.fori_loop(0, NODE_CHUNK // 16, norm_rows(bias_b), 0)
                pltpu.sync_copy(rowsb, out_hbm.at[pl.ds(r0, NODE_CHUNK), pl.ds(H_HALF, H_HALF)])

        for j in range(ncpt):
            norm_chunk(s + NS * j)

        @pl.when(s < ncrem)
        def _norm_rem():
            norm_chunk(ncpt * NS + s)

    return body(z2a, z2b, src, dst, b_gc)


def kernel(x, edge_index, W_lin, b_lin, W_gc, b_gc):
    src = edge_index[0].astype(jnp.int32)
    dst = edge_index[1].astype(jnp.int32)
    z2a, z2b = _project(x, W_lin, b_lin, W_gc)
    return _sc_aggregate(z2a, z2b, src, dst, b_gc)
